# baseline (speedup 1.0000x reference)
import functools

import jax
import jax.numpy as jnp
from jax import lax
from jax.experimental import pallas as pl
from jax.experimental.pallas import tpu as pltpu
from jax.experimental.pallas import tpu_sc as plsc

N = 10000
E = 320000
D_IN = 128
D_H = 64

NT = 10112
NBLK = NT // 128
TAIL = N - 78 * 128
EP = 327680
NCHUNK = EP // 128
CPT = NCHUNK // 16
G = 8
IDXB = 3
GDEPTH = 3
HBM_SPLIT = 3
BR = 1024
TCGRID = (N + BR - 1) // BR


def _make_prop(with_deg: bool):
  NBUF = 4 if with_deg else 5
  GD = GDEPTH if with_deg else 4
  mesh = plsc.VectorSubcoreMesh(core_axis_name="c", subcore_axis_name="s")
  f32 = jnp.float32
  out_type = [jax.ShapeDtypeStruct((N, D_H), f32)] * 4
  scratch = [
      pltpu.VMEM_SHARED((NT, D_H), f32),
      pltpu.VMEM_SHARED((NT, D_H), f32),
      pltpu.VMEM((IDXB * G, 128), jnp.int32),
      pltpu.VMEM((IDXB * G, 128), jnp.int32),
      pltpu.VMEM((NBUF * 128, D_H), f32),
      pltpu.SemaphoreType.DMA,
      pltpu.SemaphoreType.DMA,
      pltpu.SemaphoreType.DMA,
      pltpu.SemaphoreType.DMA,
      pltpu.SemaphoreType.DMA,
  ]
  if with_deg:
    out_type += [jax.ShapeDtypeStruct((N, 8), f32)] * 4
    scratch += [
        pltpu.VMEM_SHARED((NT, 8), f32),
        pltpu.VMEM((128, 8), f32),
    ]

  def body(*refs):
    (t00, t01, t10, t11,
     c00, r00, c01, r01, c10, r10, c11, r11,
     zsrc, z8src, osrc) = refs[:15]
    if with_deg:
      (a00, a01, a10, a11, d00, d01, d10, d11,
       tab, acc, cidx, ridx, vals, gsem, g2sem, ssem, dsem, isem,
       accd, oblk) = refs[15:]
    else:
      (a00, a01, a10, a11,
       tab, acc, cidx, ridx, vals, gsem, g2sem, ssem, dsem, isem) = refs[15:]
      accd = oblk = None
      d00 = d01 = d10 = d11 = None

    c = lax.axis_index("c")
    s = lax.axis_index("s")

    if with_deg:
      pltpu.sync_copy(osrc, oblk)

    def wait_gather(table_h, sem):
      pltpu.make_async_copy(
          table_h.at[pl.ds(0, 128)], vals.at[pl.ds(0, 128)], sem).wait()

    def wait_scatter(table_h):
      pltpu.make_async_copy(
          table_h.at[pl.ds(0, 128)], vals.at[pl.ds(0, 128)], ssem).wait()

    def wait_deg():
      pltpu.make_async_copy(osrc, oblk, dsem).wait()

    def wait_idx(cols2):
      pltpu.make_async_copy(
          cols2.at[pl.ds(0, G)], cidx.at[pl.ds(0, G)], isem).wait()

    def do_rel(cols2, rows2, table_h, a_out, d_out):
      def init_body(j, carry):
        b = s + j * 16
        @pl.when(b < NBLK)
        def _():
          sl = pl.ds(b * 128, 128)
          dz = pltpu.async_copy(zsrc, acc.at[sl], isem)
          dd = (pltpu.async_copy(z8src, accd.at[sl], isem)
                if with_deg else None)
          @pl.when(b < NBLK - 1)
          def _():
            pltpu.async_copy(table_h.at[sl], tab.at[sl], isem).wait()
          @pl.when(b == NBLK - 1)
          def _():
            tl = pl.ds((NBLK - 1) * 128, TAIL)
            pltpu.async_copy(table_h.at[tl], tab.at[tl], isem).wait()
          dz.wait()
          if dd is not None:
            dd.wait()
        return carry
      lax.fori_loop(0, (NBLK + 15) // 16, init_body, 0)
      plsc.subcore_barrier()

      base = s * CPT

      def vbuf(i):
        return vals.at[pl.ds((i % NBUF) * 128, 128)]

      pltpu.sync_copy(cols2.at[pl.ds(base, G)], cidx.at[pl.ds(0, G)])
      pltpu.sync_copy(rows2.at[pl.ds(base, G)], ridx.at[pl.ds(0, G)])
      pltpu.async_copy(cols2.at[pl.ds(base + G, G)], cidx.at[pl.ds(G, G)], isem)
      pltpu.async_copy(rows2.at[pl.ds(base + G, G)], ridx.at[pl.ds(G, G)], isem)

      def chunk_body(k, carry):
        @pl.when(jnp.logical_and(k % G == 0,
                                 jnp.logical_and(k > 0, k < CPT)))
        def _():
          wait_idx(cols2)
          wait_idx(cols2)
          @pl.when(k + G < CPT)
          def _():
            dst = ((k // G + 1) % IDXB) * G
            pltpu.async_copy(cols2.at[pl.ds(base + k + G, G)],
                             cidx.at[pl.ds(dst, G)], isem)
            pltpu.async_copy(rows2.at[pl.ds(base + k + G, G)],
                             ridx.at[pl.ds(dst, G)], isem)

        @pl.when(jnp.logical_and(k >= NBUF, k < CPT))
        def _():
          wait_scatter(table_h)
          if with_deg:
            wait_deg()

        @pl.when(k < CPT)
        def _():
          ci = cidx.at[k % (IDXB * G)]
          @pl.when(k % 8 < HBM_SPLIT)
          def _():
            pltpu.async_copy(table_h.at[ci], vbuf(k), g2sem)
          @pl.when(k % 8 >= HBM_SPLIT)
          def _():
            pltpu.async_copy(tab.at[ci], vbuf(k), gsem)

        i = k - (GD - 1)
        @pl.when(i >= 0)
        def _():
          @pl.when(i % 8 < HBM_SPLIT)
          def _():
            wait_gather(table_h, g2sem)
          @pl.when(i % 8 >= HBM_SPLIT)
          def _():
            wait_gather(table_h, gsem)
          r = ridx.at[i % (IDXB * G)]
          pltpu.async_copy(vbuf(i), acc.at[r], ssem, add=True)
          if with_deg:
            pltpu.async_copy(oblk, accd.at[r], dsem, add=True)
        return carry

      lax.fori_loop(0, CPT + GD - 1, chunk_body, 0)
      for _ in range(NBUF):
        wait_scatter(table_h)
        if with_deg:
          wait_deg()
      plsc.subcore_barrier()

      def out_body(j, carry):
        b = s + j * 16
        @pl.when(b < NBLK)
        def _():
          @pl.when(b < NBLK - 1)
          def _():
            sl = pl.ds(b * 128, 128)
            da = pltpu.async_copy(acc.at[sl], a_out.at[sl], isem)
            if with_deg:
              pltpu.async_copy(accd.at[sl], d_out.at[sl], isem).wait()
            da.wait()
          @pl.when(b == NBLK - 1)
          def _():
            tl = pl.ds((NBLK - 1) * 128, TAIL)
            da = pltpu.async_copy(acc.at[tl], a_out.at[tl], isem)
            if with_deg:
              pltpu.async_copy(accd.at[tl], d_out.at[tl], isem).wait()
            da.wait()
        return carry
      lax.fori_loop(0, (NBLK + 15) // 16, out_body, 0)
      plsc.subcore_barrier()

    @pl.when(c == 0)
    def _():
      do_rel(c00, r00, t00, a00, d00)
      do_rel(c01, r01, t01, a01, d01)

    @pl.when(c == 1)
    def _():
      do_rel(c10, r10, t10, a10, d10)
      do_rel(c11, r11, t11, a11, d11)

  return pl.kernel(
      body, out_type=out_type, mesh=mesh, scratch_types=scratch,
      compiler_params=pltpu.CompilerParams(use_tc_tiling_on_sc=False))


_prop_deg = _make_prop(with_deg=True)
_prop = _make_prop(with_deg=False)


def _mm4(x0, x1, wa, wb, wc, wd):
  k = x0.shape[1]
  f32 = jnp.float32

  def kern(x0r, x1r, war, wbr, wcr, wdr, o00, o01, o10, o11):
    a = x0r[...]
    b = x1r[...]
    o00[...] = jnp.dot(a, war[...], preferred_element_type=f32)
    o01[...] = jnp.dot(b, wbr[...], preferred_element_type=f32)
    o10[...] = jnp.dot(a, wcr[...], preferred_element_type=f32)
    o11[...] = jnp.dot(b, wdr[...], preferred_element_type=f32)

  xspec = pl.BlockSpec((BR, k), lambda i: (i, 0))
  wspec = pl.BlockSpec((k, D_H), lambda i: (0, 0))
  ospec = pl.BlockSpec((BR, D_H), lambda i: (i, 0))
  return pl.pallas_call(
      kern, grid=(TCGRID,),
      in_specs=[xspec, xspec, wspec, wspec, wspec, wspec],
      out_specs=[ospec] * 4,
      out_shape=[jax.ShapeDtypeStruct((N, D_H), f32)] * 4,
  )(x0, x1, wa, wb, wc, wd)


def _comb_mm(a00, a01, a10, a11, d00, d01, d10, d11, wa, wb, wc, wd):
  f32 = jnp.float32

  def kern(a00r, a01r, a10r, a11r, d0r, d1r, d2r, d3r,
           war, wbr, wcr, wdr, e0o, e1o, o00, o01, o10, o11):
    inv0 = 1.0 / jnp.maximum(d0r[...][:, 0:1], 1.0)
    inv1 = 1.0 / jnp.maximum(d1r[...][:, 0:1], 1.0)
    inv2 = 1.0 / jnp.maximum(d2r[...][:, 0:1], 1.0)
    inv3 = 1.0 / jnp.maximum(d3r[...][:, 0:1], 1.0)
    e0 = jnp.maximum(a00r[...] * inv0 + a01r[...] * inv1, 0.0)
    e1 = jnp.maximum(a10r[...] * inv2 + a11r[...] * inv3, 0.0)
    e0o[...] = e0
    e1o[...] = e1
    o00[...] = jnp.dot(e0, war[...], preferred_element_type=f32)
    o01[...] = jnp.dot(e1, wbr[...], preferred_element_type=f32)
    o10[...] = jnp.dot(e0, wcr[...], preferred_element_type=f32)
    o11[...] = jnp.dot(e1, wdr[...], preferred_element_type=f32)

  aspec = pl.BlockSpec((BR, D_H), lambda i: (i, 0))
  dspec = pl.BlockSpec((BR, 8), lambda i: (i, 0))
  wspec = pl.BlockSpec((D_H, D_H), lambda i: (0, 0))
  return pl.pallas_call(
      kern, grid=(TCGRID,),
      in_specs=[aspec] * 4 + [dspec] * 4 + [wspec] * 4,
      out_specs=[aspec] * 6,
      out_shape=[jax.ShapeDtypeStruct((N, D_H), f32)] * 6,
  )(a00, a01, a10, a11, d00, d01, d10, d11, wa, wb, wc, wd)


def _final_comb(a00, a01, a10, a11, d00, d01, d10, d11, e0_0, e0_1):
  f32 = jnp.float32

  def kern(a00r, a01r, a10r, a11r, d0r, d1r, d2r, d3r, e0r, e1r, o0, o1):
    inv0 = 1.0 / jnp.maximum(d0r[...][:, 0:1], 1.0)
    inv1 = 1.0 / jnp.maximum(d1r[...][:, 0:1], 1.0)
    inv2 = 1.0 / jnp.maximum(d2r[...][:, 0:1], 1.0)
    inv3 = 1.0 / jnp.maximum(d3r[...][:, 0:1], 1.0)
    ef0 = a00r[...] * inv0 + a01r[...] * inv1
    ef1 = a10r[...] * inv2 + a11r[...] * inv3
    e0 = e0r[...]
    e1 = e1r[...]
    o0[...] = jnp.concatenate([e0, e0, ef0], axis=1)
    o1[...] = jnp.concatenate([e1, e1, ef1], axis=1)

  aspec = pl.BlockSpec((BR, D_H), lambda i: (i, 0))
  dspec = pl.BlockSpec((BR, 8), lambda i: (i, 0))
  ospec = pl.BlockSpec((BR, 3 * D_H), lambda i: (i, 0))
  return pl.pallas_call(
      kern, grid=(TCGRID,),
      in_specs=[aspec] * 4 + [dspec] * 4 + [aspec] * 2,
      out_specs=[ospec] * 2,
      out_shape=[jax.ShapeDtypeStruct((N, 3 * D_H), f32)] * 2,
  )(a00, a01, a10, a11, d00, d01, d10, d11, e0_0, e0_1)


def _prep_edges(ei):
  rows = ei[0]
  cols = ei[1]
  pad = EP - E
  prow = (N + (jnp.arange(pad, dtype=jnp.int32) % (NT - N))).astype(jnp.int32)
  rows2 = jnp.concatenate([rows, prow]).reshape(NCHUNK, 128)
  cols2 = jnp.concatenate([cols, jnp.zeros((pad,), jnp.int32)]).reshape(NCHUNK, 128)
  return cols2, rows2


def kernel(x0, x1, ei00, ei01, ei10, ei11, W1, W2, W3, W4):
  f32 = jnp.float32
  c00, r00 = _prep_edges(ei00)
  c01, r01 = _prep_edges(ei01)
  c10, r10 = _prep_edges(ei10)
  c11, r11 = _prep_edges(ei11)
  zsrc = jnp.zeros((128, D_H), f32)
  z8src = jnp.zeros((128, 8), f32)
  osrc = jnp.ones((128, 8), f32)

  h00, h01, h10, h11 = _mm4(x0, x1, W1[0], W1[1], W1[2], W1[3])
  (a00, a01, a10, a11, d00, d01, d10, d11) = _prop_deg(
      h00, h01, h10, h11,
      c00, r00, c01, r01, c10, r10, c11, r11, zsrc, z8src, osrc)
  e0_0, e0_1, g00, g01, g10, g11 = _comb_mm(
      a00, a01, a10, a11, d00, d01, d10, d11, W3[0], W3[1], W3[2], W3[3])

  (b00, b01, b10, b11) = _prop(
      g00, g01, g10, g11,
      c00, r00, c01, r01, c10, r10, c11, r11, zsrc, z8src, osrc)
  e2_0, e2_1, f00, f01, f10, f11 = _comb_mm(
      b00, b01, b10, b11, d00, d01, d10, d11, W4[0], W4[1], W4[2], W4[3])

  (k00, k01, k10, k11) = _prop(
      f00, f01, f10, f11,
      c00, r00, c01, r01, c10, r10, c11, r11, zsrc, z8src, osrc)
  out0, out1 = _final_comb(k00, k01, k10, k11, d00, d01, d10, d11, e0_0, e0_1)
  return out0, out1


# speedup vs baseline: 1.3270x; 1.3270x over previous
import functools

import jax
import jax.numpy as jnp
from jax import lax
from jax.experimental import pallas as pl
from jax.experimental.pallas import tpu as pltpu
from jax.experimental.pallas import tpu_sc as plsc

N = 10000
E = 320000
D_IN = 128
D_H = 64

NT = 10112
NBLK = NT // 128
TAIL = N - 78 * 128
EP = 327680
NCHUNK = EP // 128
CPT = NCHUNK // 16
G = 8
IDXB = 3
GDEPTH = 3
BR = 1024
TCGRID = (N + BR - 1) // BR


def _make_prop(with_deg: bool):
  NBUF = 4 if with_deg else 5
  GD = GDEPTH if with_deg else 4
  mesh = plsc.VectorSubcoreMesh(core_axis_name="c", subcore_axis_name="s")
  f32 = jnp.float32
  out_type = [jax.ShapeDtypeStruct((N, D_H), f32)] * 4
  scratch = [
      pltpu.VMEM_SHARED((NT, D_H), f32),
      pltpu.VMEM_SHARED((NT, D_H), f32),
      pltpu.VMEM((IDXB * G, 128), jnp.int32),
      pltpu.VMEM((IDXB * G, 128), jnp.int32),
      pltpu.VMEM((NBUF * 128, D_H), f32),
      pltpu.SemaphoreType.DMA,
      pltpu.SemaphoreType.DMA,
      pltpu.SemaphoreType.DMA,
      pltpu.SemaphoreType.DMA,
  ]
  if with_deg:
    out_type += [jax.ShapeDtypeStruct((N, 8), f32)] * 4
    scratch += [
        pltpu.VMEM_SHARED((NT, 8), f32),
        pltpu.VMEM((128, 8), f32),
    ]

  def body(*refs):
    (t00, t01, t10, t11,
     c00, r00, c01, r01, c10, r10, c11, r11,
     zsrc, z8src, osrc) = refs[:15]
    if with_deg:
      (a00, a01, a10, a11, d00, d01, d10, d11,
       tab, acc, cidx, ridx, vals, gsem, ssem, dsem, isem,
       accd, oblk) = refs[15:]
    else:
      (a00, a01, a10, a11,
       tab, acc, cidx, ridx, vals, gsem, ssem, dsem, isem) = refs[15:]
      accd = oblk = None
      d00 = d01 = d10 = d11 = None

    c = lax.axis_index("c")
    s = lax.axis_index("s")

    if with_deg:
      pltpu.sync_copy(osrc, oblk)

    def wait_gather(table_h, sem):
      pltpu.make_async_copy(
          table_h.at[pl.ds(0, 128)], vals.at[pl.ds(0, 128)], sem).wait()

    def wait_scatter(table_h):
      pltpu.make_async_copy(
          table_h.at[pl.ds(0, 128)], vals.at[pl.ds(0, 128)], ssem).wait()

    def wait_deg():
      pltpu.make_async_copy(osrc, oblk, dsem).wait()

    def wait_idx(cols2):
      pltpu.make_async_copy(
          cols2.at[pl.ds(0, G)], cidx.at[pl.ds(0, G)], isem).wait()

    def do_rel(cols2, rows2, table_h, a_out, d_out):
      def init_body(j, carry):
        b = s + j * 16
        @pl.when(b < NBLK)
        def _():
          sl = pl.ds(b * 128, 128)
          dz = pltpu.async_copy(zsrc, acc.at[sl], isem)
          dd = (pltpu.async_copy(z8src, accd.at[sl], isem)
                if with_deg else None)
          @pl.when(b < NBLK - 1)
          def _():
            pltpu.async_copy(table_h.at[sl], tab.at[sl], isem).wait()
          @pl.when(b == NBLK - 1)
          def _():
            tl = pl.ds((NBLK - 1) * 128, TAIL)
            pltpu.async_copy(table_h.at[tl], tab.at[tl], isem).wait()
          dz.wait()
          if dd is not None:
            dd.wait()
        return carry
      lax.fori_loop(0, (NBLK + 15) // 16, init_body, 0)
      plsc.subcore_barrier()

      base = s * CPT

      def vbuf(i):
        return vals.at[pl.ds((i % NBUF) * 128, 128)]

      pltpu.sync_copy(cols2.at[pl.ds(base, G)], cidx.at[pl.ds(0, G)])
      pltpu.sync_copy(rows2.at[pl.ds(base, G)], ridx.at[pl.ds(0, G)])
      pltpu.async_copy(cols2.at[pl.ds(base + G, G)], cidx.at[pl.ds(G, G)], isem)
      pltpu.async_copy(rows2.at[pl.ds(base + G, G)], ridx.at[pl.ds(G, G)], isem)

      def chunk_body(k, carry):
        @pl.when(jnp.logical_and(k % G == 0,
                                 jnp.logical_and(k > 0, k < CPT)))
        def _():
          wait_idx(cols2)
          wait_idx(cols2)
          @pl.when(k + G < CPT)
          def _():
            dst = ((k // G + 1) % IDXB) * G
            pltpu.async_copy(cols2.at[pl.ds(base + k + G, G)],
                             cidx.at[pl.ds(dst, G)], isem)
            pltpu.async_copy(rows2.at[pl.ds(base + k + G, G)],
                             ridx.at[pl.ds(dst, G)], isem)

        @pl.when(jnp.logical_and(k >= NBUF, k < CPT))
        def _():
          wait_scatter(table_h)
          if with_deg:
            wait_deg()

        @pl.when(k < CPT)
        def _():
          pltpu.async_copy(tab.at[cidx.at[k % (IDXB * G)]], vbuf(k), gsem)

        i = k - (GD - 1)
        @pl.when(i >= 0)
        def _():
          wait_gather(table_h, gsem)
          r = ridx.at[i % (IDXB * G)]
          pltpu.async_copy(vbuf(i), acc.at[r], ssem, add=True)
          if with_deg:
            pltpu.async_copy(oblk, accd.at[r], dsem, add=True)
        return carry

      lax.fori_loop(0, CPT + GD - 1, chunk_body, 0)
      for _ in range(NBUF):
        wait_scatter(table_h)
        if with_deg:
          wait_deg()
      plsc.subcore_barrier()

      def out_body(j, carry):
        b = s + j * 16
        @pl.when(b < NBLK)
        def _():
          @pl.when(b < NBLK - 1)
          def _():
            sl = pl.ds(b * 128, 128)
            da = pltpu.async_copy(acc.at[sl], a_out.at[sl], isem)
            if with_deg:
              pltpu.async_copy(accd.at[sl], d_out.at[sl], isem).wait()
            da.wait()
          @pl.when(b == NBLK - 1)
          def _():
            tl = pl.ds((NBLK - 1) * 128, TAIL)
            da = pltpu.async_copy(acc.at[tl], a_out.at[tl], isem)
            if with_deg:
              pltpu.async_copy(accd.at[tl], d_out.at[tl], isem).wait()
            da.wait()
        return carry
      lax.fori_loop(0, (NBLK + 15) // 16, out_body, 0)
      plsc.subcore_barrier()

    @pl.when(c == 0)
    def _():
      do_rel(c00, r00, t00, a00, d00)
      do_rel(c01, r01, t01, a01, d01)

    @pl.when(c == 1)
    def _():
      do_rel(c10, r10, t10, a10, d10)
      do_rel(c11, r11, t11, a11, d11)

  return pl.kernel(
      body, out_type=out_type, mesh=mesh, scratch_types=scratch,
      compiler_params=pltpu.CompilerParams(use_tc_tiling_on_sc=False))


_prop_deg = _make_prop(with_deg=True)
_prop = _make_prop(with_deg=False)


def _mm4(x0, x1, wa, wb, wc, wd):
  k = x0.shape[1]
  f32 = jnp.float32

  def kern(x0r, x1r, war, wbr, wcr, wdr, o00, o01, o10, o11):
    a = x0r[...]
    b = x1r[...]
    o00[...] = jnp.dot(a, war[...], preferred_element_type=f32)
    o01[...] = jnp.dot(b, wbr[...], preferred_element_type=f32)
    o10[...] = jnp.dot(a, wcr[...], preferred_element_type=f32)
    o11[...] = jnp.dot(b, wdr[...], preferred_element_type=f32)

  xspec = pl.BlockSpec((BR, k), lambda i: (i, 0))
  wspec = pl.BlockSpec((k, D_H), lambda i: (0, 0))
  ospec = pl.BlockSpec((BR, D_H), lambda i: (i, 0))
  return pl.pallas_call(
      kern, grid=(TCGRID,),
      in_specs=[xspec, xspec, wspec, wspec, wspec, wspec],
      out_specs=[ospec] * 4,
      out_shape=[jax.ShapeDtypeStruct((N, D_H), f32)] * 4,
  )(x0, x1, wa, wb, wc, wd)


def _comb_mm(a00, a01, a10, a11, d00, d01, d10, d11, wa, wb, wc, wd):
  f32 = jnp.float32

  def kern(a00r, a01r, a10r, a11r, d0r, d1r, d2r, d3r,
           war, wbr, wcr, wdr, e0o, e1o, o00, o01, o10, o11):
    inv0 = 1.0 / jnp.maximum(d0r[...][:, 0:1], 1.0)
    inv1 = 1.0 / jnp.maximum(d1r[...][:, 0:1], 1.0)
    inv2 = 1.0 / jnp.maximum(d2r[...][:, 0:1], 1.0)
    inv3 = 1.0 / jnp.maximum(d3r[...][:, 0:1], 1.0)
    e0 = jnp.maximum(a00r[...] * inv0 + a01r[...] * inv1, 0.0)
    e1 = jnp.maximum(a10r[...] * inv2 + a11r[...] * inv3, 0.0)
    e0o[...] = e0
    e1o[...] = e1
    o00[...] = jnp.dot(e0, war[...], preferred_element_type=f32)
    o01[...] = jnp.dot(e1, wbr[...], preferred_element_type=f32)
    o10[...] = jnp.dot(e0, wcr[...], preferred_element_type=f32)
    o11[...] = jnp.dot(e1, wdr[...], preferred_element_type=f32)

  aspec = pl.BlockSpec((BR, D_H), lambda i: (i, 0))
  dspec = pl.BlockSpec((BR, 8), lambda i: (i, 0))
  wspec = pl.BlockSpec((D_H, D_H), lambda i: (0, 0))
  return pl.pallas_call(
      kern, grid=(TCGRID,),
      in_specs=[aspec] * 4 + [dspec] * 4 + [wspec] * 4,
      out_specs=[aspec] * 6,
      out_shape=[jax.ShapeDtypeStruct((N, D_H), f32)] * 6,
  )(a00, a01, a10, a11, d00, d01, d10, d11, wa, wb, wc, wd)


def _final_comb(a00, a01, a10, a11, d00, d01, d10, d11, e0_0, e0_1):
  f32 = jnp.float32

  def kern(a00r, a01r, a10r, a11r, d0r, d1r, d2r, d3r, e0r, e1r, o0, o1):
    inv0 = 1.0 / jnp.maximum(d0r[...][:, 0:1], 1.0)
    inv1 = 1.0 / jnp.maximum(d1r[...][:, 0:1], 1.0)
    inv2 = 1.0 / jnp.maximum(d2r[...][:, 0:1], 1.0)
    inv3 = 1.0 / jnp.maximum(d3r[...][:, 0:1], 1.0)
    ef0 = a00r[...] * inv0 + a01r[...] * inv1
    ef1 = a10r[...] * inv2 + a11r[...] * inv3
    e0 = e0r[...]
    e1 = e1r[...]
    o0[...] = jnp.concatenate([e0, e0, ef0], axis=1)
    o1[...] = jnp.concatenate([e1, e1, ef1], axis=1)

  aspec = pl.BlockSpec((BR, D_H), lambda i: (i, 0))
  dspec = pl.BlockSpec((BR, 8), lambda i: (i, 0))
  ospec = pl.BlockSpec((BR, 3 * D_H), lambda i: (i, 0))
  return pl.pallas_call(
      kern, grid=(TCGRID,),
      in_specs=[aspec] * 4 + [dspec] * 4 + [aspec] * 2,
      out_specs=[ospec] * 2,
      out_shape=[jax.ShapeDtypeStruct((N, 3 * D_H), f32)] * 2,
  )(a00, a01, a10, a11, d00, d01, d10, d11, e0_0, e0_1)


def _prep_edges(ei):
  rows = ei[0]
  cols = ei[1]
  pad = EP - E
  prow = (N + (jnp.arange(pad, dtype=jnp.int32) % (NT - N))).astype(jnp.int32)
  rows2 = jnp.concatenate([rows, prow]).reshape(NCHUNK, 128)
  cols2 = jnp.concatenate([cols, jnp.zeros((pad,), jnp.int32)]).reshape(NCHUNK, 128)
  return cols2, rows2


def kernel(x0, x1, ei00, ei01, ei10, ei11, W1, W2, W3, W4):
  f32 = jnp.float32
  c00, r00 = _prep_edges(ei00)
  c01, r01 = _prep_edges(ei01)
  c10, r10 = _prep_edges(ei10)
  c11, r11 = _prep_edges(ei11)
  zsrc = jnp.zeros((128, D_H), f32)
  z8src = jnp.zeros((128, 8), f32)
  osrc = jnp.ones((128, 8), f32)

  h00, h01, h10, h11 = _mm4(x0, x1, W1[0], W1[1], W1[2], W1[3])
  (a00, a01, a10, a11, d00, d01, d10, d11) = _prop_deg(
      h00, h01, h10, h11,
      c00, r00, c01, r01, c10, r10, c11, r11, zsrc, z8src, osrc)
  e0_0, e0_1, g00, g01, g10, g11 = _comb_mm(
      a00, a01, a10, a11, d00, d01, d10, d11, W3[0], W3[1], W3[2], W3[3])

  (b00, b01, b10, b11) = _prop(
      g00, g01, g10, g11,
      c00, r00, c01, r01, c10, r10, c11, r11, zsrc, z8src, osrc)
  e2_0, e2_1, f00, f01, f10, f11 = _comb_mm(
      b00, b01, b10, b11, d00, d01, d10, d11, W4[0], W4[1], W4[2], W4[3])

  (k00, k01, k10, k11) = _prop(
      f00, f01, f10, f11,
      c00, r00, c01, r01, c10, r10, c11, r11, zsrc, z8src, osrc)
  out0, out1 = _final_comb(k00, k01, k10, k11, d00, d01, d10, d11, e0_0, e0_1)
  return out0, out1


# speedup vs baseline: 1.3281x; 1.0008x over previous
import jax
import jax.numpy as jnp
from jax import lax
from jax.experimental import pallas as pl
from jax.experimental.pallas import tpu as pltpu
from jax.experimental.pallas import tpu_sc as plsc

N = 10000
E = 320000
D_IN = 128
D_H = 64

NT = 10112
NBLK = NT // 128
TAIL = N - 78 * 128
EP = 327680
NCHUNK = EP // 128
CPT = NCHUNK // 16
G = 8
IDXB = 3
GDEPTH = 3
BR = 1024
TCGRID = (N + BR - 1) // BR


def _make_prop(with_deg: bool):
  NBUF = 4 if with_deg else 5
  GD = GDEPTH if with_deg else 4
  mesh = plsc.VectorSubcoreMesh(core_axis_name="c", subcore_axis_name="s")
  f32 = jnp.float32
  out_type = [jax.ShapeDtypeStruct((N, D_H), f32)] * 4
  scratch = [
      pltpu.VMEM_SHARED((NT, D_H), f32),
      pltpu.VMEM_SHARED((NT, D_H), f32),
      pltpu.VMEM((IDXB * G, 128), jnp.int32),
      pltpu.VMEM((IDXB * G, 128), jnp.int32),
      pltpu.VMEM((NBUF * 128, D_H), f32),
      pltpu.SemaphoreType.DMA,
      pltpu.SemaphoreType.DMA,
      pltpu.SemaphoreType.DMA,
      pltpu.SemaphoreType.DMA,
  ]
  if with_deg:
    out_type += [jax.ShapeDtypeStruct((N, 8), f32)] * 4
    scratch += [
        pltpu.VMEM_SHARED((NT, 8), f32),
        pltpu.VMEM((128, 8), f32),
    ]

  def body(*refs):
    (t00, t01, t10, t11,
     c00, r00, c01, r01, c10, r10, c11, r11,
     zsrc, z8src, osrc) = refs[:15]
    if with_deg:
      (a00, a01, a10, a11, d00, d01, d10, d11,
       tab, acc, cidx, ridx, vals, gsem, ssem, dsem, isem,
       accd, oblk) = refs[15:]
    else:
      (a00, a01, a10, a11,
       tab, acc, cidx, ridx, vals, gsem, ssem, dsem, isem) = refs[15:]
      accd = oblk = None
      d00 = d01 = d10 = d11 = None

    c = lax.axis_index("c")
    s = lax.axis_index("s")

    if with_deg:
      pltpu.sync_copy(osrc, oblk)

    def wait_gather(table_h, sem):
      pltpu.make_async_copy(
          table_h.at[pl.ds(0, 128)], vals.at[pl.ds(0, 128)], sem).wait()

    def wait_scatter(table_h):
      pltpu.make_async_copy(
          table_h.at[pl.ds(0, 128)], vals.at[pl.ds(0, 128)], ssem).wait()

    def wait_deg():
      pltpu.make_async_copy(osrc, oblk, dsem).wait()

    def wait_idx(cols2):
      pltpu.make_async_copy(
          cols2.at[pl.ds(0, G)], cidx.at[pl.ds(0, G)], isem).wait()

    def do_rel(cols2, rows2, table_h, a_out, d_out):
      def init_body(j, carry):
        b = s + j * 16
        @pl.when(b < NBLK)
        def _():
          sl = pl.ds(b * 128, 128)
          dz = pltpu.async_copy(zsrc, acc.at[sl], isem)
          dd = (pltpu.async_copy(z8src, accd.at[sl], isem)
                if with_deg else None)
          @pl.when(b < NBLK - 1)
          def _():
            pltpu.async_copy(table_h.at[sl], tab.at[sl], isem).wait()
          @pl.when(b == NBLK - 1)
          def _():
            tl = pl.ds((NBLK - 1) * 128, TAIL)
            pltpu.async_copy(table_h.at[tl], tab.at[tl], isem).wait()
          dz.wait()
          if dd is not None:
            dd.wait()
        return carry
      lax.fori_loop(0, (NBLK + 15) // 16, init_body, 0)
      plsc.subcore_barrier()

      base = s * CPT

      def vbuf(i):
        return vals.at[pl.ds((i % NBUF) * 128, 128)]

      pltpu.sync_copy(cols2.at[pl.ds(base, G)], cidx.at[pl.ds(0, G)])
      pltpu.sync_copy(rows2.at[pl.ds(base, G)], ridx.at[pl.ds(0, G)])
      pltpu.async_copy(cols2.at[pl.ds(base + G, G)], cidx.at[pl.ds(G, G)], isem)
      pltpu.async_copy(rows2.at[pl.ds(base + G, G)], ridx.at[pl.ds(G, G)], isem)

      def chunk_body(k, carry):
        @pl.when(jnp.logical_and(k % G == 0,
                                 jnp.logical_and(k > 0, k < CPT)))
        def _():
          wait_idx(cols2)
          wait_idx(cols2)
          @pl.when(k + G < CPT)
          def _():
            dst = ((k // G + 1) % IDXB) * G
            pltpu.async_copy(cols2.at[pl.ds(base + k + G, G)],
                             cidx.at[pl.ds(dst, G)], isem)
            pltpu.async_copy(rows2.at[pl.ds(base + k + G, G)],
                             ridx.at[pl.ds(dst, G)], isem)

        @pl.when(jnp.logical_and(k >= NBUF, k < CPT))
        def _():
          wait_scatter(table_h)
          if with_deg:
            wait_deg()

        @pl.when(k < CPT)
        def _():
          pltpu.async_copy(tab.at[cidx.at[k % (IDXB * G)]], vbuf(k), gsem)

        i = k - (GD - 1)
        @pl.when(i >= 0)
        def _():
          wait_gather(table_h, gsem)
          r = ridx.at[i % (IDXB * G)]
          pltpu.async_copy(vbuf(i), acc.at[r], ssem, add=True)
          if with_deg:
            pltpu.async_copy(oblk, accd.at[r], dsem, add=True)
        return carry

      lax.fori_loop(0, CPT + GD - 1, chunk_body, 0)
      for _ in range(NBUF):
        wait_scatter(table_h)
        if with_deg:
          wait_deg()
      plsc.subcore_barrier()

      def out_body(j, carry):
        b = s + j * 16
        @pl.when(b < NBLK)
        def _():
          @pl.when(b < NBLK - 1)
          def _():
            sl = pl.ds(b * 128, 128)
            da = pltpu.async_copy(acc.at[sl], a_out.at[sl], isem)
            if with_deg:
              pltpu.async_copy(accd.at[sl], d_out.at[sl], isem).wait()
            da.wait()
          @pl.when(b == NBLK - 1)
          def _():
            tl = pl.ds((NBLK - 1) * 128, TAIL)
            da = pltpu.async_copy(acc.at[tl], a_out.at[tl], isem)
            if with_deg:
              pltpu.async_copy(accd.at[tl], d_out.at[tl], isem).wait()
            da.wait()
        return carry
      lax.fori_loop(0, (NBLK + 15) // 16, out_body, 0)
      plsc.subcore_barrier()

    @pl.when(c == 0)
    def _():
      do_rel(c00, r00, t00, a00, d00)
      do_rel(c01, r01, t01, a01, d01)

    @pl.when(c == 1)
    def _():
      do_rel(c10, r10, t10, a10, d10)
      do_rel(c11, r11, t11, a11, d11)

  return pl.kernel(
      body, out_type=out_type, mesh=mesh, scratch_types=scratch,
      compiler_params=pltpu.CompilerParams(use_tc_tiling_on_sc=False))


_prop_deg = _make_prop(with_deg=True)
_prop = _make_prop(with_deg=False)


def _mm4(x0, x1, wa, wb, wc, wd):
  k = x0.shape[1]
  f32 = jnp.float32

  def kern(x0r, x1r, war, wbr, wcr, wdr, o00, o01, o10, o11):
    a = x0r[...]
    b = x1r[...]
    o00[...] = jnp.dot(a, war[...], preferred_element_type=f32)
    o01[...] = jnp.dot(b, wbr[...], preferred_element_type=f32)
    o10[...] = jnp.dot(a, wcr[...], preferred_element_type=f32)
    o11[...] = jnp.dot(b, wdr[...], preferred_element_type=f32)

  xspec = pl.BlockSpec((BR, k), lambda i: (i, 0))
  wspec = pl.BlockSpec((k, D_H), lambda i: (0, 0))
  ospec = pl.BlockSpec((BR, D_H), lambda i: (i, 0))
  return pl.pallas_call(
      kern, grid=(TCGRID,),
      in_specs=[xspec, xspec, wspec, wspec, wspec, wspec],
      out_specs=[ospec] * 4,
      out_shape=[jax.ShapeDtypeStruct((N, D_H), f32)] * 4,
  )(x0, x1, wa, wb, wc, wd)


def _comb_mm(a00, a01, a10, a11, d00, d01, d10, d11, wa, wb, wc, wd):
  f32 = jnp.float32

  def kern(a00r, a01r, a10r, a11r, d0r, d1r, d2r, d3r,
           war, wbr, wcr, wdr, e0o, e1o, o00, o01, o10, o11):
    inv0 = 1.0 / jnp.maximum(d0r[...][:, 0:1], 1.0)
    inv1 = 1.0 / jnp.maximum(d1r[...][:, 0:1], 1.0)
    inv2 = 1.0 / jnp.maximum(d2r[...][:, 0:1], 1.0)
    inv3 = 1.0 / jnp.maximum(d3r[...][:, 0:1], 1.0)
    e0 = jnp.maximum(a00r[...] * inv0 + a01r[...] * inv1, 0.0)
    e1 = jnp.maximum(a10r[...] * inv2 + a11r[...] * inv3, 0.0)
    e0o[...] = e0
    e1o[...] = e1
    o00[...] = jnp.dot(e0, war[...], preferred_element_type=f32)
    o01[...] = jnp.dot(e1, wbr[...], preferred_element_type=f32)
    o10[...] = jnp.dot(e0, wcr[...], preferred_element_type=f32)
    o11[...] = jnp.dot(e1, wdr[...], preferred_element_type=f32)

  aspec = pl.BlockSpec((BR, D_H), lambda i: (i, 0))
  dspec = pl.BlockSpec((BR, 8), lambda i: (i, 0))
  wspec = pl.BlockSpec((D_H, D_H), lambda i: (0, 0))
  return pl.pallas_call(
      kern, grid=(TCGRID,),
      in_specs=[aspec] * 4 + [dspec] * 4 + [wspec] * 4,
      out_specs=[aspec] * 6,
      out_shape=[jax.ShapeDtypeStruct((N, D_H), f32)] * 6,
  )(a00, a01, a10, a11, d00, d01, d10, d11, wa, wb, wc, wd)


def _final_comb(a00, a01, a10, a11, d00, d01, d10, d11, e0_0, e0_1):
  f32 = jnp.float32

  def kern(a00r, a01r, a10r, a11r, d0r, d1r, d2r, d3r, e0r, e1r, o0, o1):
    inv0 = 1.0 / jnp.maximum(d0r[...][:, 0:1], 1.0)
    inv1 = 1.0 / jnp.maximum(d1r[...][:, 0:1], 1.0)
    inv2 = 1.0 / jnp.maximum(d2r[...][:, 0:1], 1.0)
    inv3 = 1.0 / jnp.maximum(d3r[...][:, 0:1], 1.0)
    ef0 = a00r[...] * inv0 + a01r[...] * inv1
    ef1 = a10r[...] * inv2 + a11r[...] * inv3
    e0 = e0r[...]
    e1 = e1r[...]
    o0[...] = jnp.concatenate([e0, e0, ef0], axis=1)
    o1[...] = jnp.concatenate([e1, e1, ef1], axis=1)

  aspec = pl.BlockSpec((BR, D_H), lambda i: (i, 0))
  dspec = pl.BlockSpec((BR, 8), lambda i: (i, 0))
  ospec = pl.BlockSpec((BR, 3 * D_H), lambda i: (i, 0))
  return pl.pallas_call(
      kern, grid=(TCGRID,),
      in_specs=[aspec] * 4 + [dspec] * 4 + [aspec] * 2,
      out_specs=[ospec] * 2,
      out_shape=[jax.ShapeDtypeStruct((N, 3 * D_H), f32)] * 2,
  )(a00, a01, a10, a11, d00, d01, d10, d11, e0_0, e0_1)


def _prep_edges(ei):
  rows = ei[0]
  cols = ei[1]
  pad = EP - E
  prow = (N + (jnp.arange(pad, dtype=jnp.int32) % (NT - N))).astype(jnp.int32)
  rows2 = jnp.concatenate([rows, prow]).reshape(NCHUNK, 128)
  cols2 = jnp.concatenate([cols, jnp.zeros((pad,), jnp.int32)]).reshape(NCHUNK, 128)
  return cols2, rows2


def kernel(x0, x1, ei00, ei01, ei10, ei11, W1, W2, W3, W4):
  f32 = jnp.float32
  c00, r00 = _prep_edges(ei00)
  c01, r01 = _prep_edges(ei01)
  c10, r10 = _prep_edges(ei10)
  c11, r11 = _prep_edges(ei11)
  zsrc = jnp.zeros((128, D_H), f32)
  z8src = jnp.zeros((128, 8), f32)
  osrc = jnp.ones((128, 8), f32)

  h00, h01, h10, h11 = _mm4(x0, x1, W1[0], W1[1], W1[2], W1[3])
  (a00, a01, a10, a11, d00, d01, d10, d11) = _prop_deg(
      h00, h01, h10, h11,
      c00, r00, c01, r01, c10, r10, c11, r11, zsrc, z8src, osrc)
  e0_0, e0_1, g00, g01, g10, g11 = _comb_mm(
      a00, a01, a10, a11, d00, d01, d10, d11, W3[0], W3[1], W3[2], W3[3])

  (b00, b01, b10, b11) = _prop(
      g00, g01, g10, g11,
      c00, r00, c01, r01, c10, r10, c11, r11, zsrc, z8src, osrc)
  e2_0, e2_1, f00, f01, f10, f11 = _comb_mm(
      b00, b01, b10, b11, d00, d01, d10, d11, W4[0], W4[1], W4[2], W4[3])

  (k00, k01, k10, k11) = _prop(
      f00, f01, f10, f11,
      c00, r00, c01, r01, c10, r10, c11, r11, zsrc, z8src, osrc)
  out0, out1 = _final_comb(k00, k01, k10, k11, d00, d01, d10, d11, e0_0, e0_1)
  return out0, out1
